# Initial kernel scaffold; baseline (speedup 1.0000x reference)
import jax
import jax.numpy as jnp
from jax.experimental import pallas as pl


def kernel(x, Wq, Wk, Wv, Wo, Wff1, bff1, Wff2, bff2, ln1_g, ln1_b, ln2_g, ln2_b):
    raise NotImplementedError("write your pallas kernel here")


# speedup vs baseline: 6.7374x; 6.7374x over previous
import functools

import numpy as np
import jax
import jax.numpy as jnp
from jax.experimental import pallas as pl
from jax.experimental.pallas import tpu as pltpu

D_MODEL = 768
HEADS = 12
DH = 64
D_FF = 512
S = 4096
U = 45
UP = 48
NKB = 32
BQ = 512
BR = 512

_INTERPRET = False

_rng = np.random.default_rng(0)
_IDX = _rng.integers(0, S, size=(S, U)).astype(np.int32)
_R48 = np.zeros((S, UP), np.int32)
_R48[:, :U] = _IDX % 128
_KB48 = np.full((S, UP), NKB, np.int32)
_KB48[:, :U] = _IDX // 128


def _proj_body(x_ref, wq_ref, wk_ref, wv_ref, q_ref, k_ref, v_ref):
    x = x_ref[...]
    for w_ref, o_ref in ((wq_ref, q_ref), (wk_ref, k_ref), (wv_ref, v_ref)):
        y = jnp.dot(x, w_ref[...], preferred_element_type=jnp.float32)
        for h in range(HEADS):
            o_ref[h, :, :] = y[:, h * DH:(h + 1) * DH]


def _measure_body(q_ref, k_ref, r_ref, kb_ref, m_ref):
    s_blk = jax.lax.dot_general(
        q_ref[0], k_ref[0], (((1,), (1,)), ((), ())),
        preferred_element_type=jnp.float32)
    s3 = s_blk.reshape(BQ, NKB, 128)
    r3 = jnp.broadcast_to(r_ref[...][:, None, :], (BQ, NKB, UP))
    g = jnp.take_along_axis(s3, r3, axis=2)
    kbsel = kb_ref[...][:, None, :] == jax.lax.broadcasted_iota(
        jnp.int32, (BQ, NKB, UP), 1)
    vals = jnp.sum(jnp.where(kbsel, g, 0.0), axis=1)
    lane = jax.lax.broadcasted_iota(jnp.int32, (BQ, UP), 1)
    valid = lane < U
    mx = jnp.max(jnp.where(valid, vals, -jnp.inf), axis=1)
    mn = jnp.sum(jnp.where(valid, vals, 0.0), axis=1) * (1.0 / U)
    m_ref[0, 0, :] = mx - mn


def _topk_body(m_ref, top_ref, scr):
    scr[...] = m_ref[0]
    iota = jax.lax.broadcasted_iota(jnp.int32, (1, S), 1)
    lane64 = jax.lax.broadcasted_iota(jnp.int32, (1, 64), 1)

    def step(u, acc):
        row = scr[...]
        m = jnp.max(row)
        idx = jnp.min(jnp.where(row == m, iota, jnp.int32(2**30)))
        scr[...] = jnp.where(iota == idx, -jnp.inf, row)
        return jnp.where(lane64 == u, idx, acc)

    top_ref[0] = jax.lax.fori_loop(0, U, step, jnp.zeros((1, 64), jnp.int32))


def _attn_body(top_smem, q_ref, k_ref, v_ref, out_ref, qsel):
    h = pl.program_id(0)
    for u in range(U):
        i = top_smem[h * 64 + u]
        qsel[pl.ds(u, 1), :] = q_ref[0, pl.ds(i, 1), :]
    scores = jax.lax.dot_general(
        qsel[...], k_ref[0], (((1,), (1,)), ((), ())),
        preferred_element_type=jnp.float32) * (1.0 / 8.0)
    smax = jnp.max(scores, axis=1, keepdims=True)
    e = jnp.exp(scores - smax)
    att = e / jnp.sum(e, axis=1, keepdims=True)
    ctx = jnp.dot(att, v_ref[0], preferred_element_type=jnp.float32)
    vmean = jnp.mean(v_ref[0], axis=0, keepdims=True)
    out_ref[0] = jnp.broadcast_to(vmean, (S, DH))
    for u in range(U):
        i = top_smem[h * 64 + u]
        out_ref[0, pl.ds(i, 1), :] = ctx[u:u + 1, :]


def _ln(y, g, b):
    mu = jnp.mean(y, axis=-1, keepdims=True)
    var = jnp.mean((y - mu) ** 2, axis=-1, keepdims=True)
    return (y - mu) / jnp.sqrt(var + 1e-3) * g + b


def _epilogue_body(ctx_ref, x_ref, wo_ref, wff1_ref, bff1_ref, wff2_ref,
                   bff2_ref, g1_ref, b1_ref, g2_ref, b2_ref, out_ref):
    ctx = jnp.concatenate([ctx_ref[h] for h in range(HEADS)], axis=1)
    attn = jnp.dot(ctx, wo_ref[...], preferred_element_type=jnp.float32)
    h1 = _ln(x_ref[...] + attn, g1_ref[...], b1_ref[...])
    ffa = jnp.maximum(
        jnp.dot(h1, wff1_ref[...], preferred_element_type=jnp.float32)
        + bff1_ref[...], 0.0)
    ff = jnp.dot(ffa, wff2_ref[...], preferred_element_type=jnp.float32) + bff2_ref[...]
    out_ref[...] = _ln(h1 + ff, g2_ref[...], b2_ref[...])


def kernel(x, Wq, Wk, Wv, Wo, Wff1, bff1, Wff2, bff2, ln1_g, ln1_b, ln2_g, ln2_b):
    B = x.shape[0]
    x2 = x.reshape(S, D_MODEL)

    q, k, v = pl.pallas_call(
        _proj_body,
        grid=(S // BR,),
        in_specs=[
            pl.BlockSpec((BR, D_MODEL), lambda i: (i, 0)),
            pl.BlockSpec((D_MODEL, D_MODEL), lambda i: (0, 0)),
            pl.BlockSpec((D_MODEL, D_MODEL), lambda i: (0, 0)),
            pl.BlockSpec((D_MODEL, D_MODEL), lambda i: (0, 0)),
        ],
        out_specs=[pl.BlockSpec((HEADS, BR, DH), lambda i: (0, i, 0))] * 3,
        out_shape=[jax.ShapeDtypeStruct((HEADS, S, DH), jnp.float32)] * 3,
        interpret=_INTERPRET,
    )(x2, Wq, Wk, Wv)

    r48 = jnp.asarray(_R48)
    kb48 = jnp.asarray(_KB48)
    m = pl.pallas_call(
        _measure_body,
        grid=(HEADS, S // BQ),
        in_specs=[
            pl.BlockSpec((1, BQ, DH), lambda h, i: (h, i, 0)),
            pl.BlockSpec((1, S, DH), lambda h, i: (h, 0, 0)),
            pl.BlockSpec((BQ, UP), lambda h, i: (i, 0)),
            pl.BlockSpec((BQ, UP), lambda h, i: (i, 0)),
        ],
        out_specs=pl.BlockSpec((1, 1, BQ), lambda h, i: (h, 0, i)),
        out_shape=jax.ShapeDtypeStruct((HEADS, 1, S), jnp.float32),
        interpret=_INTERPRET,
    )(q, k, r48, kb48)

    m_top = pl.pallas_call(
        _topk_body,
        grid=(HEADS,),
        in_specs=[pl.BlockSpec((1, 1, S), lambda h: (h, 0, 0))],
        out_specs=pl.BlockSpec((1, 1, 64), lambda h: (h, 0, 0)),
        out_shape=jax.ShapeDtypeStruct((HEADS, 1, 64), jnp.int32),
        scratch_shapes=[pltpu.VMEM((1, S), jnp.float32)],
        interpret=_INTERPRET,
    )(m)

    ctx = pl.pallas_call(
        _attn_body,
        grid_spec=pltpu.PrefetchScalarGridSpec(
            num_scalar_prefetch=1,
            grid=(HEADS,),
            in_specs=[
                pl.BlockSpec((1, S, DH), lambda h, *_: (h, 0, 0)),
                pl.BlockSpec((1, S, DH), lambda h, *_: (h, 0, 0)),
                pl.BlockSpec((1, S, DH), lambda h, *_: (h, 0, 0)),
            ],
            out_specs=pl.BlockSpec((1, S, DH), lambda h, *_: (h, 0, 0)),
            scratch_shapes=[pltpu.VMEM((UP, DH), jnp.float32)],
        ),
        out_shape=jax.ShapeDtypeStruct((HEADS, S, DH), jnp.float32),
        interpret=_INTERPRET,
    )(m_top.reshape(-1), q, k, v)

    out = pl.pallas_call(
        _epilogue_body,
        grid=(S // BR,),
        in_specs=[
            pl.BlockSpec((HEADS, BR, DH), lambda i: (0, i, 0)),
            pl.BlockSpec((BR, D_MODEL), lambda i: (i, 0)),
            pl.BlockSpec((D_MODEL, D_MODEL), lambda i: (0, 0)),
            pl.BlockSpec((D_MODEL, D_FF), lambda i: (0, 0)),
            pl.BlockSpec((1, D_FF), lambda i: (0, 0)),
            pl.BlockSpec((D_FF, D_MODEL), lambda i: (0, 0)),
            pl.BlockSpec((1, D_MODEL), lambda i: (0, 0)),
            pl.BlockSpec((1, D_MODEL), lambda i: (0, 0)),
            pl.BlockSpec((1, D_MODEL), lambda i: (0, 0)),
            pl.BlockSpec((1, D_MODEL), lambda i: (0, 0)),
            pl.BlockSpec((1, D_MODEL), lambda i: (0, 0)),
        ],
        out_specs=pl.BlockSpec((BR, D_MODEL), lambda i: (i, 0)),
        out_shape=jax.ShapeDtypeStruct((S, D_MODEL), jnp.float32),
        interpret=_INTERPRET,
    )(ctx, x2, Wo, Wff1, bff1.reshape(1, -1), Wff2, bff2.reshape(1, -1),
      ln1_g.reshape(1, -1), ln1_b.reshape(1, -1),
      ln2_g.reshape(1, -1), ln2_b.reshape(1, -1))

    return out.reshape(B, S, D_MODEL)
